# baseline (speedup 1.0000x reference)
import functools

import jax
import jax.numpy as jnp
from jax import lax
from jax.experimental import pallas as pl
from jax.experimental.pallas import tpu as pltpu
from jax.experimental.pallas import tpu_sc as plsc

NC = 2
NS = 16
LANES = 16
NW = NC * NS


def kernel(input_ids, word_emb, pos_emb):
    B, S = input_ids.shape
    V, D = word_emb.shape
    N = B * S
    P = S // NW
    CP = 8
    SLOTS = 3
    LEAD = 2
    n_chunks = P // CP

    if input_ids.dtype != jnp.int32:
        input_ids = input_ids.astype(jnp.int32)
    mesh = plsc.VectorSubcoreMesh(
        core_axis_name="c", subcore_axis_name="s",
        num_cores=NC, num_subcores=NS)

    @functools.partial(
        pl.kernel,
        out_type=jax.ShapeDtypeStruct((B, S, D), jnp.float32),
        mesh=mesh,
        scratch_types=[
            pltpu.VMEM((B, P), jnp.int32),
            pltpu.VMEM((SLOTS, CP, D), jnp.float32),
            pltpu.VMEM((SLOTS, B, CP, D), jnp.float32),
        ] + [pltpu.SemaphoreType.DMA] * (2 * SLOTS + 1),
    )
    def emb_kernel(ids_hbm, tab_hbm, pos_hbm, out_hbm, idx_v, pos_s, rows_s,
                   *sems):
        g_sem = sems[:SLOTS]
        o_sem = sems[SLOTS:2 * SLOTS]
        i_sem = sems[2 * SLOTS]
        cid = lax.axis_index("c")
        sid = lax.axis_index("s")
        wid = sid * NC + cid
        pos0 = wid * P

        idx_descs = [pltpu.make_async_copy(
            ids_hbm.at[b, pl.ds(pos0, P)], idx_v.at[b], i_sem)
            for b in range(B)]

        def gather_descs(ci, slot):
            ds = [pltpu.make_async_copy(
                pos_hbm.at[pl.ds(pos0 + ci * CP, CP)], pos_s.at[slot],
                g_sem[slot])]
            for b in range(B):
                ds.append(pltpu.make_async_copy(
                    tab_hbm.at[idx_v.at[b, pl.ds(ci * CP, CP)]],
                    rows_s.at[slot, b], g_sem[slot]))
            return ds

        def out_descs(ci, slot):
            return [pltpu.make_async_copy(
                rows_s.at[slot],
                out_hbm.at[:, pl.ds(pos0 + ci * CP, CP)],
                o_sem[slot])]

        def start_all(descs):
            for d_ in descs:
                d_.start()

        def wait_all(descs):
            for d_ in descs:
                d_.wait()

        start_all(idx_descs)
        for ci in range(LEAD):
            gather_descs(ci, ci % SLOTS)[0].start()
        wait_all(idx_descs)
        for ci in range(LEAD):
            start_all(gather_descs(ci, ci % SLOTS)[1:])

        def chunk_body(ci, carry):
            slot_t = lax.rem(ci, SLOTS)
            for k in range(SLOTS):
                @pl.when(slot_t == k)
                def _():
                    wait_all(gather_descs(ci, k))

            @plsc.parallel_loop(0, CP)
            def _(r):
                for sl_i in range(D // LANES):
                    sl = pl.ds(sl_i * LANES, LANES)
                    pvec = pos_s[slot_t, r, sl]
                    for b in range(B):
                        plsc.addupdate(rows_s.at[slot_t, b, r, sl], pvec)

            for k in range(SLOTS):
                @pl.when(slot_t == k)
                def _():
                    start_all(out_descs(ci, k))

            nxt = ci + LEAD
            nslot_t = lax.rem(nxt, SLOTS)

            @pl.when(nxt < n_chunks)
            def _():
                for k in range(SLOTS):
                    @pl.when(nslot_t == k)
                    def _():
                        @pl.when(ci >= 1)
                        def _():
                            wait_all(out_descs(ci - 1, k))
                        start_all(gather_descs(nxt, k))
            return carry

        lax.fori_loop(0, n_chunks, chunk_body, 0)

        for ci in range(n_chunks - SLOTS, n_chunks):
            wait_all(out_descs(ci, ci % SLOTS))

    return emb_kernel(input_ids, word_emb, pos_emb)


# speedup vs baseline: 1.0165x; 1.0165x over previous
import functools

import jax
import jax.numpy as jnp
from jax import lax
from jax.experimental import pallas as pl
from jax.experimental.pallas import tpu as pltpu
from jax.experimental.pallas import tpu_sc as plsc

NC = 2
NS = 16
LANES = 16
NW = NC * NS


def kernel(input_ids, word_emb, pos_emb):
    B, S = input_ids.shape
    V, D = word_emb.shape
    N = B * S
    P = S // NW
    CP = 8
    SLOTS = 3
    LEAD = 2
    n_chunks = P // CP

    if input_ids.dtype != jnp.int32:
        input_ids = input_ids.astype(jnp.int32)
    mesh = plsc.VectorSubcoreMesh(
        core_axis_name="c", subcore_axis_name="s",
        num_cores=NC, num_subcores=NS)

    @functools.partial(
        pl.kernel,
        out_type=jax.ShapeDtypeStruct((B, S, D), jnp.float32),
        mesh=mesh,
        scratch_types=[
            pltpu.VMEM((B, P), jnp.int32),
            pltpu.VMEM((SLOTS, CP, D), jnp.float32),
            pltpu.VMEM((SLOTS, B, CP, D), jnp.float32),
        ] + [pltpu.SemaphoreType.DMA] * (2 * SLOTS + 1),
    )
    def emb_kernel(ids_hbm, tab_hbm, pos_hbm, out_hbm, idx_v, pos_s, rows_s,
                   *sems):
        g_sem = sems[:SLOTS]
        o_sem = sems[SLOTS:2 * SLOTS]
        i_sem = sems[2 * SLOTS]
        cid = lax.axis_index("c")
        sid = lax.axis_index("s")
        wid = sid * NC + cid
        pos0 = wid * P

        idx_descs = [pltpu.make_async_copy(
            ids_hbm.at[b, pl.ds(pos0, P)], idx_v.at[b], i_sem)
            for b in range(B)]

        def gather_descs(ci, slot):
            ds = [pltpu.make_async_copy(
                pos_hbm.at[pl.ds(pos0 + ci * CP, CP)], pos_s.at[slot],
                g_sem[slot])]
            for b in range(B):
                ds.append(pltpu.make_async_copy(
                    tab_hbm.at[idx_v.at[b, pl.ds(ci * CP, CP)]],
                    rows_s.at[slot, b], g_sem[slot]))
            return ds

        def out_descs(ci, slot):
            return [pltpu.make_async_copy(
                rows_s.at[slot],
                out_hbm.at[:, pl.ds(pos0 + ci * CP, CP)],
                o_sem[slot])]

        def start_all(descs):
            for d_ in descs:
                d_.start()

        def wait_all(descs):
            for d_ in descs:
                d_.wait()

        start_all(idx_descs)
        for ci in range(LEAD):
            gather_descs(ci, ci % SLOTS)[0].start()
        wait_all(idx_descs)
        for ci in range(LEAD):
            start_all(gather_descs(ci, ci % SLOTS)[1:])

        def chunk_body(ci, carry):
            slot_t = lax.rem(ci, SLOTS)
            for k in range(SLOTS):
                @pl.when(slot_t == k)
                def _():
                    wait_all(gather_descs(ci, k))

            def row_add(r, carry2):
                for sl_i in range(D // LANES):
                    sl = pl.ds(sl_i * LANES, LANES)
                    pvec = pos_s[slot_t, r, sl]
                    for b in range(B):
                        plsc.addupdate(rows_s.at[slot_t, b, r, sl], pvec)
                return carry2

            lax.fori_loop(0, CP, row_add, 0)

            for k in range(SLOTS):
                @pl.when(slot_t == k)
                def _():
                    start_all(out_descs(ci, k))

            nxt = ci + LEAD
            nslot_t = lax.rem(nxt, SLOTS)

            @pl.when(nxt < n_chunks)
            def _():
                for k in range(SLOTS):
                    @pl.when(nslot_t == k)
                    def _():
                        @pl.when(ci >= 1)
                        def _():
                            wait_all(out_descs(ci - 1, k))
                        start_all(gather_descs(nxt, k))
            return carry

        lax.fori_loop(0, n_chunks, chunk_body, 0)

        for ci in range(n_chunks - SLOTS, n_chunks):
            wait_all(out_descs(ci, ci % SLOTS))

    return emb_kernel(input_ids, word_emb, pos_emb)
